# baseline (speedup 1.0000x reference)
import functools
import math

import jax
import jax.numpy as jnp
from jax import lax
from jax.experimental import pallas as pl
from jax.experimental.pallas import tpu as pltpu
from jax.experimental.pallas import tpu_sc as plsc


def _router_body(cap, ns, x_ref, wr_ref, br_ref, dest_ref, wn_ref,
                 carry_ref):
    b = pl.program_id(0)
    tb = x_ref.shape[0]
    e = wr_ref.shape[1]

    @pl.when(b == 0)
    def _():
        carry_ref[...] = jnp.zeros_like(carry_ref)

    logits = jnp.dot(x_ref[...], wr_ref[...],
                     precision=lax.Precision.HIGHEST,
                     preferred_element_type=jnp.float32) + br_ref[...]
    m = jnp.max(logits, axis=1, keepdims=True)
    s = jnp.sum(jnp.exp(logits - m), axis=1, keepdims=True)
    p = 1.0 / s
    t = p / (p + 1e-9)
    wn = t / (t + 1e-9)

    col = lax.broadcasted_iota(jnp.int32, (tb, e), 1)
    e_idx = jnp.min(jnp.where(logits == m, col, e), axis=1, keepdims=True)

    onehot = (col == e_idx).astype(jnp.float32)
    ii = lax.broadcasted_iota(jnp.int32, (tb, tb), 0)
    jj = lax.broadcasted_iota(jnp.int32, (tb, tb), 1)
    lstrict = (jj < ii).astype(jnp.float32)
    csum = jnp.dot(lstrict, onehot, preferred_element_type=jnp.float32)
    pos = jnp.sum(onehot * (csum + carry_ref[...]), axis=1, keepdims=True)
    carry_ref[...] += jnp.sum(onehot, axis=0, keepdims=True)

    pos_i = pos.astype(jnp.int32)
    keep = pos_i < cap
    dest_ref[...] = jnp.where(keep, e_idx * cap + pos_i, ns)
    wn_ref[...] = jnp.where(keep, wn, 0.0)


def _router(x2, wr, br, cap, nblk):
    n, d = x2.shape
    e = wr.shape[1]
    tb = n // nblk
    return pl.pallas_call(
        functools.partial(_router_body, cap, e * cap),
        grid=(nblk,),
        in_specs=[
            pl.BlockSpec((tb, d), lambda b: (b, 0)),
            pl.BlockSpec((d, e), lambda b: (0, 0)),
            pl.BlockSpec((1, e), lambda b: (0, 0)),
        ],
        out_specs=[
            pl.BlockSpec((tb, 1), lambda b: (b, 0)),
            pl.BlockSpec((tb, 1), lambda b: (b, 0)),
        ],
        out_shape=[
            jax.ShapeDtypeStruct((n, 1), jnp.int32),
            jax.ShapeDtypeStruct((n, 1), jnp.float32),
        ],
        scratch_shapes=[pltpu.VMEM((1, e), jnp.float32)],
    )(x2, wr, br.reshape(1, e))


def _meta_body(sb, dest_ref, wn_ref, src_ref, wns_ref):
    s = pl.program_id(0)
    n = dest_ref.shape[0]
    slot = lax.broadcasted_iota(jnp.int32, (n, sb), 1) + s * sb
    cmp = (dest_ref[...] == slot).astype(jnp.float32)
    ids = lax.broadcasted_iota(jnp.int32, (n, 1), 0).astype(jnp.float32)
    src = lax.dot_general(cmp, ids, (((0,), (0,)), ((), ())),
                          precision=lax.Precision.HIGHEST,
                          preferred_element_type=jnp.float32)
    wns = lax.dot_general(cmp, wn_ref[...], (((0,), (0,)), ((), ())),
                          precision=lax.Precision.HIGHEST,
                          preferred_element_type=jnp.float32)
    src_ref[...] = src.astype(jnp.int32)
    wns_ref[...] = wns


def _meta(dest, wn, ns, nblk):
    n = dest.shape[0]
    sb = ns // nblk
    return pl.pallas_call(
        functools.partial(_meta_body, sb),
        grid=(nblk,),
        in_specs=[
            pl.BlockSpec((n, 1), lambda s: (0, 0)),
            pl.BlockSpec((n, 1), lambda s: (0, 0)),
        ],
        out_specs=[
            pl.BlockSpec((sb, 1), lambda s: (s, 0)),
            pl.BlockSpec((sb, 1), lambda s: (s, 0)),
        ],
        out_shape=[
            jax.ShapeDtypeStruct((ns, 1), jnp.int32),
            jax.ShapeDtypeStruct((ns, 1), jnp.float32),
        ],
    )(dest, wn)


def _sc_gather(table, idx):
    nrows = idx.shape[0]
    d = table.shape[1]
    nc, nsub = 2, 16
    nw = nc * nsub
    rpw = nrows // nw
    mesh = plsc.VectorSubcoreMesh(core_axis_name="c", subcore_axis_name="s",
                                  num_cores=nc, num_subcores=nsub)

    @functools.partial(
        pl.kernel,
        out_type=jax.ShapeDtypeStruct((nrows, d), jnp.float32),
        mesh=mesh,
        scratch_types=[
            pltpu.VMEM((rpw,), jnp.int32),
            pltpu.VMEM((rpw, d), jnp.float32),
            pltpu.SemaphoreType.DMA,
        ],
    )
    def k(table_hbm, idx_hbm, out_hbm, idx_v, rows_v, sem):
        wid = lax.axis_index("s") * nc + lax.axis_index("c")
        base = wid * rpw
        pltpu.sync_copy(idx_hbm.at[pl.ds(base, rpw)], idx_v)
        pltpu.async_copy(table_hbm.at[idx_v], rows_v, sem).wait()
        pltpu.sync_copy(rows_v, out_hbm.at[pl.ds(base, rpw)])

    return k(table, idx)


def _ffn_body(ne, ein_ref, w1_ref, b1_ref, w2_ref, b2_ref, wns_ref,
              eo_ref):
    i = pl.program_id(0)

    @pl.when(i == ne)
    def _():
        eo_ref[0] = jnp.zeros_like(eo_ref[0])

    @pl.when(i < ne)
    def _():
        h = jnp.dot(ein_ref[0], w1_ref[0],
                    preferred_element_type=jnp.float32) + b1_ref[0]
        g = 0.5 * h * (1.0 + lax.erf(h * (1.0 / math.sqrt(2.0))))
        out = jnp.dot(g, w2_ref[0],
                      preferred_element_type=jnp.float32) + b2_ref[0]
        eo_ref[0] = out * wns_ref[0]


def _ffn(ein3, w1, b1, w2, b2, wns3):
    e, cap, d = ein3.shape
    dff = w1.shape[2]
    last = e - 1

    def wmap(i):
        return (jnp.minimum(i, last), 0, 0)

    return pl.pallas_call(
        functools.partial(_ffn_body, e),
        grid=(e + 1,),
        in_specs=[
            pl.BlockSpec((1, cap, d), wmap),
            pl.BlockSpec((1, d, dff), wmap),
            pl.BlockSpec((1, 1, dff), wmap),
            pl.BlockSpec((1, dff, d), wmap),
            pl.BlockSpec((1, 1, d), wmap),
            pl.BlockSpec((1, cap, 1), wmap),
        ],
        out_specs=pl.BlockSpec((1, cap, d), lambda i: (i, 0, 0)),
        out_shape=jax.ShapeDtypeStruct((e + 1, cap, d), jnp.float32),
        compiler_params=pltpu.CompilerParams(
            dimension_semantics=("arbitrary",)),
    )(ein3, w1, b1.reshape(e, 1, dff), w2, b2.reshape(e, 1, d), wns3)


def kernel(x, Wr, br, W1, b1, W2, b2):
    orig_shape = x.shape
    d = orig_shape[-1]
    x2 = x.reshape(-1, d)
    n = x2.shape[0]
    e = Wr.shape[1]
    cap = max(1, int(math.ceil(float(n) / float(e))))
    ns = e * cap

    dest, wn = _router(x2, Wr, br, cap, nblk=8)
    src, wns = _meta(dest, wn, ns, nblk=8)
    ein = _sc_gather(x2, src.reshape(ns))
    eo = _ffn(ein.reshape(e, cap, d), W1, b1, W2, b2,
              wns.reshape(e, cap, 1))
    y = _sc_gather(eo.reshape((e + 1) * cap, d), dest.reshape(n))
    return y.reshape(orig_shape)


# speedup vs baseline: 1.0053x; 1.0053x over previous
import functools
import math

import jax
import jax.numpy as jnp
from jax import lax
from jax.experimental import pallas as pl
from jax.experimental.pallas import tpu as pltpu
from jax.experimental.pallas import tpu_sc as plsc


def _router_body(cap, ns, x_ref, wr_ref, br_ref, dest_ref, wn_ref,
                 carry_ref):
    b = pl.program_id(0)
    tb = x_ref.shape[0]
    e = wr_ref.shape[1]

    @pl.when(b == 0)
    def _():
        carry_ref[...] = jnp.zeros_like(carry_ref)

    logits = jnp.dot(x_ref[...], wr_ref[...],
                     preferred_element_type=jnp.float32) + br_ref[...]
    m = jnp.max(logits, axis=1, keepdims=True)
    s = jnp.sum(jnp.exp(logits - m), axis=1, keepdims=True)
    p = 1.0 / s
    t = p / (p + 1e-9)
    wn = t / (t + 1e-9)

    col = lax.broadcasted_iota(jnp.int32, (tb, e), 1)
    e_idx = jnp.min(jnp.where(logits == m, col, e), axis=1, keepdims=True)

    onehot = (col == e_idx).astype(jnp.float32)
    ii = lax.broadcasted_iota(jnp.int32, (tb, tb), 0)
    jj = lax.broadcasted_iota(jnp.int32, (tb, tb), 1)
    lstrict = (jj < ii).astype(jnp.float32)
    csum = jnp.dot(lstrict, onehot, preferred_element_type=jnp.float32)
    pos = jnp.sum(onehot * (csum + carry_ref[...]), axis=1, keepdims=True)
    carry_ref[...] += jnp.sum(onehot, axis=0, keepdims=True)

    pos_i = pos.astype(jnp.int32)
    keep = pos_i < cap
    dest_ref[...] = jnp.where(keep, e_idx * cap + pos_i, ns)
    wn_ref[...] = jnp.where(keep, wn, 0.0)


def _router(x2, wr, br, cap, nblk):
    n, d = x2.shape
    e = wr.shape[1]
    tb = n // nblk
    return pl.pallas_call(
        functools.partial(_router_body, cap, e * cap),
        grid=(nblk,),
        in_specs=[
            pl.BlockSpec((tb, d), lambda b: (b, 0)),
            pl.BlockSpec((d, e), lambda b: (0, 0)),
            pl.BlockSpec((1, e), lambda b: (0, 0)),
        ],
        out_specs=[
            pl.BlockSpec((tb, 1), lambda b: (b, 0)),
            pl.BlockSpec((tb, 1), lambda b: (b, 0)),
        ],
        out_shape=[
            jax.ShapeDtypeStruct((n, 1), jnp.int32),
            jax.ShapeDtypeStruct((n, 1), jnp.float32),
        ],
        scratch_shapes=[pltpu.VMEM((1, e), jnp.float32)],
    )(x2, wr, br.reshape(1, e))


def _meta_body(sb, dest_ref, wn_ref, src_ref, wns_ref):
    s = pl.program_id(0)
    n = dest_ref.shape[0]
    slot = lax.broadcasted_iota(jnp.int32, (n, sb), 1) + s * sb
    cmp = (dest_ref[...] == slot).astype(jnp.float32)
    ids = lax.broadcasted_iota(jnp.int32, (n, 1), 0).astype(jnp.float32)
    src = lax.dot_general(cmp, ids, (((0,), (0,)), ((), ())),
                          precision=lax.Precision.HIGHEST,
                          preferred_element_type=jnp.float32)
    wns = lax.dot_general(cmp, wn_ref[...], (((0,), (0,)), ((), ())),
                          precision=lax.Precision.HIGHEST,
                          preferred_element_type=jnp.float32)
    src_ref[...] = src.astype(jnp.int32)
    wns_ref[...] = wns


def _meta(dest, wn, ns, nblk):
    n = dest.shape[0]
    sb = ns // nblk
    return pl.pallas_call(
        functools.partial(_meta_body, sb),
        grid=(nblk,),
        in_specs=[
            pl.BlockSpec((n, 1), lambda s: (0, 0)),
            pl.BlockSpec((n, 1), lambda s: (0, 0)),
        ],
        out_specs=[
            pl.BlockSpec((sb, 1), lambda s: (s, 0)),
            pl.BlockSpec((sb, 1), lambda s: (s, 0)),
        ],
        out_shape=[
            jax.ShapeDtypeStruct((ns, 1), jnp.int32),
            jax.ShapeDtypeStruct((ns, 1), jnp.float32),
        ],
    )(dest, wn)


def _sc_gather(table, idx):
    nrows = idx.shape[0]
    d = table.shape[1]
    nc, nsub = 2, 16
    nw = nc * nsub
    rpw = nrows // nw
    mesh = plsc.VectorSubcoreMesh(core_axis_name="c", subcore_axis_name="s",
                                  num_cores=nc, num_subcores=nsub)

    @functools.partial(
        pl.kernel,
        out_type=jax.ShapeDtypeStruct((nrows, d), jnp.float32),
        mesh=mesh,
        scratch_types=[
            pltpu.VMEM((rpw,), jnp.int32),
            pltpu.VMEM((rpw, d), jnp.float32),
            pltpu.SemaphoreType.DMA,
        ],
    )
    def k(table_hbm, idx_hbm, out_hbm, idx_v, rows_v, sem):
        wid = lax.axis_index("s") * nc + lax.axis_index("c")
        base = wid * rpw
        pltpu.sync_copy(idx_hbm.at[pl.ds(base, rpw)], idx_v)
        pltpu.async_copy(table_hbm.at[idx_v], rows_v, sem).wait()
        pltpu.sync_copy(rows_v, out_hbm.at[pl.ds(base, rpw)])

    return k(table, idx)


def _ffn_body(ne, ein_ref, w1_ref, b1_ref, w2_ref, b2_ref, wns_ref,
              eo_ref):
    i = pl.program_id(0)

    @pl.when(i == ne)
    def _():
        eo_ref[0] = jnp.zeros_like(eo_ref[0])

    @pl.when(i < ne)
    def _():
        h = jnp.dot(ein_ref[0], w1_ref[0],
                    preferred_element_type=jnp.float32) + b1_ref[0]
        g = 0.5 * h * (1.0 + lax.erf(h * (1.0 / math.sqrt(2.0))))
        out = jnp.dot(g, w2_ref[0],
                      preferred_element_type=jnp.float32) + b2_ref[0]
        eo_ref[0] = out * wns_ref[0]


def _ffn(ein3, w1, b1, w2, b2, wns3):
    e, cap, d = ein3.shape
    dff = w1.shape[2]
    last = e - 1

    def wmap(i):
        return (jnp.minimum(i, last), 0, 0)

    return pl.pallas_call(
        functools.partial(_ffn_body, e),
        grid=(e + 1,),
        in_specs=[
            pl.BlockSpec((1, cap, d), wmap),
            pl.BlockSpec((1, d, dff), wmap),
            pl.BlockSpec((1, 1, dff), wmap),
            pl.BlockSpec((1, dff, d), wmap),
            pl.BlockSpec((1, 1, d), wmap),
            pl.BlockSpec((1, cap, 1), wmap),
        ],
        out_specs=pl.BlockSpec((1, cap, d), lambda i: (i, 0, 0)),
        out_shape=jax.ShapeDtypeStruct((e + 1, cap, d), jnp.float32),
        compiler_params=pltpu.CompilerParams(
            dimension_semantics=("arbitrary",)),
    )(ein3, w1, b1.reshape(e, 1, dff), w2, b2.reshape(e, 1, d), wns3)


def kernel(x, Wr, br, W1, b1, W2, b2):
    orig_shape = x.shape
    d = orig_shape[-1]
    x2 = x.reshape(-1, d)
    n = x2.shape[0]
    e = Wr.shape[1]
    cap = max(1, int(math.ceil(float(n) / float(e))))
    ns = e * cap

    dest, wn = _router(x2, Wr, br, cap, nblk=8)
    src, wns = _meta(dest, wn, ns, nblk=8)
    ein = _sc_gather(x2, src.reshape(ns))
    eo = _ffn(ein.reshape(e, cap, d), W1, b1, W2, b2,
              wns.reshape(e, cap, 1))
    y = _sc_gather(eo.reshape((e + 1) * cap, d), dest.reshape(n))
    return y.reshape(orig_shape)


# speedup vs baseline: 1.1701x; 1.1639x over previous
import functools
import math

import jax
import jax.numpy as jnp
from jax import lax
from jax.experimental import pallas as pl
from jax.experimental.pallas import tpu as pltpu


def _moe_body(ne, nf, cap, nblk, x_ref, wr_ref, br_ref, w1_ref, b1_ref,
              w2_ref, b2_ref, y_ref, dest_ref, wn_ref, ein_ref, eo_ref):
    i = pl.program_id(0)
    n, d = x_ref.shape
    e_num = wr_ref.shape[1]
    ns = ne * cap
    tb = n // nblk

    @pl.when(i == 0)
    def _router():
        carry = jnp.zeros((1, e_num), jnp.float32)
        for b in range(nblk):
            xb = x_ref[b * tb:(b + 1) * tb, :]
            logits = jnp.dot(xb, wr_ref[...],
                             preferred_element_type=jnp.float32) + br_ref[...]
            m = jnp.max(logits, axis=1, keepdims=True)
            s = jnp.sum(jnp.exp(logits - m), axis=1, keepdims=True)
            p = 1.0 / s
            t = p / (p + 1e-9)
            wn = t / (t + 1e-9)

            col = lax.broadcasted_iota(jnp.int32, (tb, e_num), 1)
            e_idx = jnp.min(jnp.where(logits == m, col, e_num),
                            axis=1, keepdims=True)
            onehot = (col == e_idx).astype(jnp.float32)
            ii = lax.broadcasted_iota(jnp.int32, (tb, tb), 0)
            jj = lax.broadcasted_iota(jnp.int32, (tb, tb), 1)
            lstrict = (jj < ii).astype(jnp.float32)
            csum = jnp.dot(lstrict, onehot,
                           preferred_element_type=jnp.float32)
            pos = jnp.sum(onehot * (csum + carry), axis=1, keepdims=True)
            carry = carry + jnp.sum(onehot, axis=0, keepdims=True)

            pos_i = pos.astype(jnp.int32)
            keep = pos_i < cap
            dest_ref[b * tb:(b + 1) * tb, :] = jnp.where(
                keep, e_idx * cap + pos_i, ns)
            wn_ref[b * tb:(b + 1) * tb, :] = jnp.where(keep, wn, 0.0)

    @pl.when((i >= 1) & (i <= ne * nf))
    def _expert():
        e = (i - 1) // nf
        f = (i - 1) % nf

        @pl.when(f == 0)
        def _():
            slot = lax.broadcasted_iota(jnp.int32, (n, cap), 1) + e * cap
            pt = (dest_ref[...] == slot).astype(jnp.float32)
            ein_ref[...] = lax.dot_general(
                pt, x_ref[...], (((0,), (0,)), ((), ())),
                preferred_element_type=jnp.float32)

        h = jnp.dot(ein_ref[...], w1_ref[0],
                    preferred_element_type=jnp.float32) + b1_ref[0]
        g = 0.5 * h * (1.0 + lax.erf(h * (1.0 / math.sqrt(2.0))))
        part = jnp.dot(g, w2_ref[0], preferred_element_type=jnp.float32)

        @pl.when(f == 0)
        def _():
            eo_ref[pl.ds(e * cap, cap), :] = part

        @pl.when(f == nf - 1)
        def _():
            if nf == 1:
                eo_ref[pl.ds(e * cap, cap), :] = part + b2_ref[0]
            else:
                eo_ref[pl.ds(e * cap, cap), :] += part + b2_ref[0]

    @pl.when(i == ne * nf + 1)
    def _combine():
        for b in range(nblk):
            db = dest_ref[b * tb:(b + 1) * tb, :]
            wb = wn_ref[b * tb:(b + 1) * tb, :]
            slot = lax.broadcasted_iota(jnp.int32, (tb, ns), 1)
            cm = (db == slot).astype(jnp.float32) * wb
            y_ref[b * tb:(b + 1) * tb, :] = jnp.dot(
                cm, eo_ref[...], preferred_element_type=jnp.float32)


def _moe(x2, wr, br, w1, b1, w2, b2, cap, tf, nblk):
    n, d = x2.shape
    e, _, dff = w1.shape
    nf = dff // tf
    nsteps = 1 + e * nf + 1
    last_e = e - 1
    last_f = nf - 1

    def we(i):
        return jnp.clip(i - 1, 0, e * nf - 1) // nf

    def wf(i):
        return jnp.clip(i - 1, 0, e * nf - 1) % nf

    return pl.pallas_call(
        functools.partial(_moe_body, e, nf, cap, nblk),
        grid=(nsteps,),
        in_specs=[
            pl.BlockSpec((n, d), lambda i: (0, 0)),
            pl.BlockSpec((d, e), lambda i: (0, 0)),
            pl.BlockSpec((1, e), lambda i: (0, 0)),
            pl.BlockSpec((1, d, tf), lambda i: (we(i), 0, wf(i))),
            pl.BlockSpec((1, 1, tf), lambda i: (we(i), 0, wf(i))),
            pl.BlockSpec((1, tf, d), lambda i: (we(i), wf(i), 0)),
            pl.BlockSpec((1, 1, d), lambda i: (we(i), 0, 0)),
        ],
        out_specs=pl.BlockSpec((n, d), lambda i: (0, 0)),
        out_shape=jax.ShapeDtypeStruct((n, d), jnp.float32),
        scratch_shapes=[
            pltpu.VMEM((n, 1), jnp.int32),
            pltpu.VMEM((n, 1), jnp.float32),
            pltpu.VMEM((cap, d), jnp.float32),
            pltpu.VMEM((e * cap, d), jnp.float32),
        ],
        compiler_params=pltpu.CompilerParams(
            dimension_semantics=("arbitrary",)),
    )(x2, wr, br.reshape(1, e), w1, b1.reshape(e, 1, dff), w2,
      b2.reshape(e, 1, d))


def kernel(x, Wr, br, W1, b1, W2, b2):
    orig_shape = x.shape
    d = orig_shape[-1]
    x2 = x.reshape(-1, d)
    n = x2.shape[0]
    e = Wr.shape[1]
    cap = max(1, int(math.ceil(float(n) / float(e))))

    y = _moe(x2, Wr, br, W1, b1, W2, b2, cap, tf=1536, nblk=8)
    return y.reshape(orig_shape)


# speedup vs baseline: 1.2175x; 1.0404x over previous
import functools
import math

import jax
import jax.numpy as jnp
from jax import lax
from jax.experimental import pallas as pl
from jax.experimental.pallas import tpu as pltpu


def _moe_body(ne, cap, nblk, x_ref, wr_ref, br_ref, w1_ref, b1_ref,
              w2_ref, b2_ref, eo_ref, dest_ref, wn_ref):
    i = pl.program_id(0)
    n, d = x_ref.shape
    e_num = wr_ref.shape[1]
    ns = ne * cap
    tb = n // nblk

    @pl.when(i == 0)
    def _router():
        carry = jnp.zeros((1, e_num), jnp.float32)
        for b in range(nblk):
            xb = x_ref[b * tb:(b + 1) * tb, :]
            logits = jnp.dot(xb, wr_ref[...],
                             preferred_element_type=jnp.float32) + br_ref[...]
            m = jnp.max(logits, axis=1, keepdims=True)
            s = jnp.sum(jnp.exp(logits - m), axis=1, keepdims=True)
            p = 1.0 / s
            t = p / (p + 1e-9)
            wn = t / (t + 1e-9)

            col = lax.broadcasted_iota(jnp.int32, (tb, e_num), 1)
            e_idx = jnp.min(jnp.where(logits == m, col, e_num),
                            axis=1, keepdims=True)
            onehot = (col == e_idx).astype(jnp.float32)
            ii = lax.broadcasted_iota(jnp.int32, (tb, tb), 0)
            jj = lax.broadcasted_iota(jnp.int32, (tb, tb), 1)
            lstrict = (jj < ii).astype(jnp.float32)
            csum = jnp.dot(lstrict, onehot,
                           preferred_element_type=jnp.float32)
            pos = jnp.sum(onehot * (csum + carry), axis=1, keepdims=True)
            carry = carry + jnp.sum(onehot, axis=0, keepdims=True)

            pos_i = pos.astype(jnp.int32)
            keep = pos_i < cap
            dest_ref[b * tb:(b + 1) * tb, :] = jnp.where(
                keep, e_idx * cap + pos_i, ns)
            wn_ref[b * tb:(b + 1) * tb, :] = jnp.where(keep, wn, 0.0)

    @pl.when(i >= 1)
    def _expert():
        e = i - 1
        slot = lax.broadcasted_iota(jnp.int32, (n, cap), 1) + e * cap
        pt = (dest_ref[...] == slot).astype(jnp.float32)
        ein = lax.dot_general(
            pt, x_ref[...], (((0,), (0,)), ((), ())),
            preferred_element_type=jnp.float32)

        h = jnp.dot(ein, w1_ref[0],
                    preferred_element_type=jnp.float32) + b1_ref[0]
        g = 0.5 * h * (1.0 + lax.erf(h * (1.0 / math.sqrt(2.0))))
        eo_ref[0] = jnp.dot(g, w2_ref[0],
                            preferred_element_type=jnp.float32) + b2_ref[0]


def _moe(x2, wr, br, w1, b1, w2, b2, cap, nblk):
    n, d = x2.shape
    e, _, dff = w1.shape

    def we(i):
        return (jnp.clip(i - 1, 0, e - 1), 0, 0)

    return pl.pallas_call(
        functools.partial(_moe_body, e, cap, nblk),
        grid=(e + 1,),
        in_specs=[
            pl.BlockSpec((n, d), lambda i: (0, 0)),
            pl.BlockSpec((d, e), lambda i: (0, 0)),
            pl.BlockSpec((1, e), lambda i: (0, 0)),
            pl.BlockSpec((1, d, dff), we),
            pl.BlockSpec((1, 1, dff), we),
            pl.BlockSpec((1, dff, d), we),
            pl.BlockSpec((1, 1, d), we),
        ],
        out_specs=[
            pl.BlockSpec((1, cap, d), we),
            pl.BlockSpec((n, 1), lambda i: (0, 0)),
            pl.BlockSpec((n, 1), lambda i: (0, 0)),
        ],
        out_shape=[
            jax.ShapeDtypeStruct((e, cap, d), jnp.float32),
            jax.ShapeDtypeStruct((n, 1), jnp.int32),
            jax.ShapeDtypeStruct((n, 1), jnp.float32),
        ],
        compiler_params=pltpu.CompilerParams(
            dimension_semantics=("arbitrary",)),
    )(x2, wr, br.reshape(1, e), w1, b1.reshape(e, 1, dff), w2,
      b2.reshape(e, 1, d))


def _combine_body(dest_ref, wn_ref, eo_ref, y_ref):
    ns = eo_ref.shape[0]
    tb = dest_ref.shape[0]
    slot = lax.broadcasted_iota(jnp.int32, (tb, ns), 1)
    cm = (dest_ref[...] == slot).astype(jnp.float32) * wn_ref[...]
    y_ref[...] = jnp.dot(cm, eo_ref[...], preferred_element_type=jnp.float32)


def _combine(dest, wn, eo2, nblk):
    ns, d = eo2.shape
    n = dest.shape[0]
    tb = n // nblk
    return pl.pallas_call(
        _combine_body,
        grid=(nblk,),
        in_specs=[
            pl.BlockSpec((tb, 1), lambda b: (b, 0)),
            pl.BlockSpec((tb, 1), lambda b: (b, 0)),
            pl.BlockSpec((ns, d), lambda b: (0, 0)),
        ],
        out_specs=pl.BlockSpec((tb, d), lambda b: (b, 0)),
        out_shape=jax.ShapeDtypeStruct((n, d), jnp.float32),
    )(dest, wn, eo2)


def kernel(x, Wr, br, W1, b1, W2, b2):
    orig_shape = x.shape
    d = orig_shape[-1]
    x2 = x.reshape(-1, d)
    n = x2.shape[0]
    e = Wr.shape[1]
    cap = max(1, int(math.ceil(float(n) / float(e))))

    eo, dest, wn = _moe(x2, Wr, br, W1, b1, W2, b2, cap, nblk=8)
    y = _combine(dest, wn, eo.reshape(e * cap, d), nblk=8)
    return y.reshape(orig_shape)


# speedup vs baseline: 1.2224x; 1.0040x over previous
import functools
import math

import jax
import jax.numpy as jnp
from jax import lax
from jax.experimental import pallas as pl
from jax.experimental.pallas import tpu as pltpu


def _moe_body(ne, cap, nblk, x_ref, wr_ref, br_ref, w1_ref, b1_ref,
              w2_ref, b2_ref, eo_ref, dest_ref, wn_ref):
    i = pl.program_id(0)
    n, d = x_ref.shape
    e_num = wr_ref.shape[1]
    ns = ne * cap
    tb = n // nblk

    @pl.when(i == 0)
    def _router():
        carry = jnp.zeros((1, e_num), jnp.float32)
        for b in range(nblk):
            xb = x_ref[b * tb:(b + 1) * tb, :]
            logits = jnp.dot(xb, wr_ref[...],
                             preferred_element_type=jnp.float32) + br_ref[...]
            m = jnp.max(logits, axis=1, keepdims=True)
            s = jnp.sum(jnp.exp(logits - m), axis=1, keepdims=True)
            p = 1.0 / s
            t = p / (p + 1e-9)
            wn = t / (t + 1e-9)

            col = lax.broadcasted_iota(jnp.int32, (tb, e_num), 1)
            e_idx = jnp.min(jnp.where(logits == m, col, e_num),
                            axis=1, keepdims=True)
            onehot = (col == e_idx).astype(jnp.float32)
            ii = lax.broadcasted_iota(jnp.int32, (tb, tb), 0)
            jj = lax.broadcasted_iota(jnp.int32, (tb, tb), 1)
            lstrict = (jj < ii).astype(jnp.float32)
            csum = jnp.dot(lstrict, onehot,
                           preferred_element_type=jnp.float32)
            pos = jnp.sum(onehot * (csum + carry), axis=1, keepdims=True)
            carry = carry + jnp.sum(onehot, axis=0, keepdims=True)

            pos_i = pos.astype(jnp.int32)
            keep = pos_i < cap
            dest_ref[b * tb:(b + 1) * tb, :] = jnp.where(
                keep, e_idx * cap + pos_i, ns)
            wn_ref[b * tb:(b + 1) * tb, :] = jnp.where(keep, wn, 0.0)

    @pl.when(i >= 1)
    def _expert():
        e = i - 1
        slot = lax.broadcasted_iota(jnp.int32, (n, cap), 1) + e * cap
        pt = (dest_ref[...] == slot).astype(jnp.float32)
        ein = lax.dot_general(
            pt, x_ref[...], (((0,), (0,)), ((), ())),
            preferred_element_type=jnp.float32)

        h = jnp.dot(ein, w1_ref[0],
                    preferred_element_type=jnp.float32) + b1_ref[0]
        g = 0.5 * h * (1.0 + lax.erf(h * (1.0 / math.sqrt(2.0))))
        eo_ref[0] = jnp.dot(g, w2_ref[0],
                            preferred_element_type=jnp.float32) + b2_ref[0]


def _moe(x2, wr, br, w1, b1, w2, b2, cap, nblk):
    n, d = x2.shape
    e, _, dff = w1.shape

    def we(i):
        return (jnp.clip(i - 1, 0, e - 1), 0, 0)

    return pl.pallas_call(
        functools.partial(_moe_body, e, cap, nblk),
        grid=(e + 1,),
        in_specs=[
            pl.BlockSpec((n, d), lambda i: (0, 0)),
            pl.BlockSpec((d, e), lambda i: (0, 0)),
            pl.BlockSpec((1, e), lambda i: (0, 0)),
            pl.BlockSpec((1, d, dff), we),
            pl.BlockSpec((1, 1, dff), we),
            pl.BlockSpec((1, dff, d), we),
            pl.BlockSpec((1, 1, d), we),
        ],
        out_specs=[
            pl.BlockSpec((1, cap, d), we),
            pl.BlockSpec((n, 1), lambda i: (0, 0)),
            pl.BlockSpec((n, 1), lambda i: (0, 0)),
        ],
        out_shape=[
            jax.ShapeDtypeStruct((e, cap, d), jnp.float32),
            jax.ShapeDtypeStruct((n, 1), jnp.int32),
            jax.ShapeDtypeStruct((n, 1), jnp.float32),
        ],
        compiler_params=pltpu.CompilerParams(
            dimension_semantics=("arbitrary",)),
    )(x2, wr, br.reshape(1, e), w1, b1.reshape(e, 1, dff), w2,
      b2.reshape(e, 1, d))


def _combine_body(dest_ref, wn_ref, eo_ref, y_ref):
    ns = eo_ref.shape[0]
    tb = dest_ref.shape[0]
    slot = lax.broadcasted_iota(jnp.int32, (tb, ns), 1)
    cm = (dest_ref[...] == slot).astype(jnp.float32) * wn_ref[...]
    y_ref[...] = jnp.dot(cm, eo_ref[...], preferred_element_type=jnp.float32)


def _combine(dest, wn, eo2, nblk):
    ns, d = eo2.shape
    n = dest.shape[0]
    tb = n // nblk
    return pl.pallas_call(
        _combine_body,
        grid=(nblk,),
        in_specs=[
            pl.BlockSpec((tb, 1), lambda b: (b, 0)),
            pl.BlockSpec((tb, 1), lambda b: (b, 0)),
            pl.BlockSpec((ns, d), lambda b: (0, 0)),
        ],
        out_specs=pl.BlockSpec((tb, d), lambda b: (b, 0)),
        out_shape=jax.ShapeDtypeStruct((n, d), jnp.float32),
    )(dest, wn, eo2)


def kernel(x, Wr, br, W1, b1, W2, b2):
    orig_shape = x.shape
    d = orig_shape[-1]
    x2 = x.reshape(-1, d)
    n = x2.shape[0]
    e = Wr.shape[1]
    cap = max(1, int(math.ceil(float(n) / float(e))))

    eo, dest, wn = _moe(x2, Wr, br, W1, b1, W2, b2, cap, nblk=8)
    y = _combine(dest, wn, eo.reshape(e * cap, d), nblk=4)
    return y.reshape(orig_shape)
